# baseline (speedup 1.0000x reference)
import functools

import jax
import jax.numpy as jnp
from jax import lax
from jax.experimental import pallas as pl
from jax.experimental.pallas import tpu as pltpu
from jax.experimental.pallas import tpu_sc as plsc

EMBED = 64
LANES = 16
NC, NS = 2, 16
NW = NC * NS
BATCH = 16
CTX = 2048
TOTAL = BATCH * CTX
BPW = TOTAL // NW
CHUNK = 128
NCHUNK = BPW // CHUNK
SCALE = 8.0


def _sc_embed(x2d, table, pos2d):
    mesh = plsc.VectorSubcoreMesh(core_axis_name="c", subcore_axis_name="s")

    @functools.partial(
        pl.kernel,
        out_type=jax.ShapeDtypeStruct((TOTAL, EMBED), jnp.float32),
        mesh=mesh,
        scratch_types=[
            pltpu.VMEM((NCHUNK, CHUNK), jnp.int32),
            pltpu.VMEM((CHUNK, EMBED), jnp.float32),
            pltpu.VMEM((CHUNK, EMBED), jnp.float32),
            pltpu.SemaphoreType.DMA,
        ],
        compiler_params=pltpu.CompilerParams(use_tc_tiling_on_sc=False),
    )
    def k(x_hbm, tab_hbm, pos_hbm, out_hbm, idx_v, rows_v, pos_v, sem):
        wid = lax.axis_index("s") * NC + lax.axis_index("c")
        base = wid * BPW
        t0 = (wid % 2) * BPW

        pltpu.sync_copy(x_hbm.at[pl.ds(wid * NCHUNK, NCHUNK)], idx_v)

        for j in range(NCHUNK):
            pltpu.async_copy(tab_hbm.at[idx_v.at[j]], rows_v, sem).wait()
            pltpu.sync_copy(pos_hbm.at[pl.ds(t0 + j * CHUNK, CHUNK)], pos_v)

            def row_body(r, _):
                for c in range(EMBED // LANES):
                    sl = (r, pl.ds(c * LANES, LANES))
                    rows_v[sl] = rows_v[sl] * SCALE + pos_v[sl]
                return 0

            lax.fori_loop(0, CHUNK, row_body, 0)
            pltpu.sync_copy(rows_v, out_hbm.at[pl.ds(base + j * CHUNK, CHUNK)])

    return k(x2d, table, pos2d)


def kernel(X, tok_table, pos_embedding):
    x2d = X.reshape(TOTAL // CHUNK, CHUNK)
    pos2d = pos_embedding.reshape(CTX, EMBED)
    out = _sc_embed(x2d, tok_table, pos2d)
    return out.reshape(BATCH, CTX, EMBED)
